# baseline (speedup 1.0000x reference)
import functools

import jax
import jax.numpy as jnp
from jax import lax
from jax.experimental import pallas as pl
from jax.experimental.pallas import tpu as pltpu
from jax.experimental.pallas import tpu_sc as plsc

N_BINS = 1000
HIDDEN = 64
MIN_VAL = -5.0
SCALE = 100.0
HP_LEN = 1024

_info = plsc.get_sparse_core_info()
_NC, _NS = _info.num_cores, _info.num_subcores
_NW = _NC * _NS

SUPER = 3200


def _make_sc_call(n_rows, row_len):
    rows_per_w = n_rows // _NW
    pair = 2 * row_len
    pairs_per_super = SUPER // pair
    n_super = rows_per_w * row_len // SUPER
    n_blocks = pair // 16
    a_done_blk = (row_len + 15) // 16

    mesh = plsc.VectorSubcoreMesh(core_axis_name="c", subcore_axis_name="s")

    @functools.partial(
        pl.kernel,
        mesh=mesh,
        compiler_params=pltpu.CompilerParams(
            needs_layout_passes=False, use_tc_tiling_on_sc=True
        ),
        out_type=jax.ShapeDtypeStruct((n_rows, row_len, HIDDEN), jnp.float32),
        scratch_types=[
            pltpu.VMEM((SUPER,), jnp.float32),
            pltpu.VMEM((2, 1, row_len, HIDDEN), jnp.float32),
            pltpu.VMEM((N_BINS * HIDDEN,), jnp.float32),
            pltpu.VMEM((HP_LEN,), jnp.float32),
            pltpu.SemaphoreType.DMA,
        ],
    )
    def sc_kernel(x_hbm, hp_hbm, table_hbm, out_hbm, x_v, rows_v, tab_v, hp_v,
                  ssem):
        wid = lax.axis_index("s") * _NC + lax.axis_index("c")
        wrow = wid * rows_per_w
        webase = wid * rows_per_w * row_len

        pltpu.sync_copy(hp_hbm, hp_v)
        pltpu.sync_copy(table_hbm, tab_v)

        def wait_store(buf):
            pltpu.make_async_copy(
                rows_v.at[buf], out_hbm.at[pl.ds(0, 1)], ssem
            ).wait()

        def pair_body(s, p):
            gp = s * pairs_per_super + p
            poff = p * pair

            def blocks(lo, hi):
                for blk in range(lo, hi):
                    xv = x_v[pl.ds(poff + blk * 16, 16)]
                    t0 = (xv - MIN_VAL) * SCALE
                    gi = jnp.clip(t0.astype(jnp.int32), 0, N_BINS - 1)
                    hi_b = plsc.load_gather(hp_v, [gi + 1])
                    lo_b = plsc.load_gather(hp_v, [gi])
                    gi = (gi + jnp.where(xv > hi_b, 1, 0)
                          - jnp.where(xv <= lo_b, 1, 0))
                    gofs = gi * HIDDEN
                    for e in range(16):
                        ep = blk * 16 + e
                        buf, row = (0, ep) if ep < row_len else (1, ep - row_len)
                        base = gofs[e]
                        for k in range(0, HIDDEN, 16):
                            rows_v[buf, 0, row, pl.ds(k, 16)] = (
                                tab_v[pl.ds(base + k, 16)]
                            )

            pl.when(gp >= 1)(functools.partial(wait_store, 0))
            blocks(0, a_done_blk - 1)
            pl.when(gp >= 1)(functools.partial(wait_store, 1))
            blocks(a_done_blk - 1, a_done_blk)
            pltpu.async_copy(
                rows_v.at[0], out_hbm.at[pl.ds(wrow + gp * 2, 1)], ssem
            )
            blocks(a_done_blk, n_blocks)
            pltpu.async_copy(
                rows_v.at[1], out_hbm.at[pl.ds(wrow + gp * 2 + 1, 1)], ssem
            )

        def super_body(s, carry):
            pltpu.sync_copy(
                x_hbm.at[pl.ds(pl.multiple_of(webase + s * SUPER, SUPER), SUPER)],
                x_v,
            )

            def inner(p, c):
                pair_body(s, p)
                return c

            lax.fori_loop(0, pairs_per_super, inner, 0)
            return carry

        lax.fori_loop(0, n_super, super_body, 0)

        wait_store(0)
        wait_store(1)

    return sc_kernel


def kernel(x, boundaries, table):
    n_rows, row_len = x.shape
    xf = x.reshape(n_rows * row_len)
    hp = jnp.concatenate(
        [
            jnp.full((1,), -jnp.inf, jnp.float32),
            boundaries.astype(jnp.float32),
            jnp.full((HP_LEN - 1 - boundaries.shape[0],), jnp.inf, jnp.float32),
        ]
    )
    tab_flat = table.reshape(N_BINS * HIDDEN)
    return _make_sc_call(n_rows, row_len)(xf, hp, tab_flat)


# speedup vs baseline: 1.2840x; 1.2840x over previous
import functools

import jax
import jax.numpy as jnp
from jax import lax
from jax.experimental import pallas as pl
from jax.experimental.pallas import tpu as pltpu
from jax.experimental.pallas import tpu_sc as plsc

N_BINS = 1000
HIDDEN = 64
MIN_VAL = -5.0
SCALE = 100.0
HP_LEN = 1024

_info = plsc.get_sparse_core_info()
_NC, _NS = _info.num_cores, _info.num_subcores
_NW = _NC * _NS

CHUNK = 128

_GDN = lax.GatherDimensionNumbers(
    offset_dims=(), collapsed_slice_dims=(0,), start_index_map=(0,)
)


def _lane_broadcast(vec, e):
    idx = jnp.full((16, 1), e, dtype=jnp.int32)
    return lax.gather(
        vec, idx, dimension_numbers=_GDN, slice_sizes=(1,),
        mode=lax.GatherScatterMode.PROMISE_IN_BOUNDS,
    )


def _make_sc_call(total):
    per_w = total // _NW
    n_chunks = per_w // CHUNK
    n_groups = n_chunks // 2

    mesh = plsc.VectorSubcoreMesh(core_axis_name="c", subcore_axis_name="s")

    @functools.partial(
        pl.kernel,
        mesh=mesh,
        compiler_params=pltpu.CompilerParams(
            needs_layout_passes=False, use_tc_tiling_on_sc=True
        ),
        out_type=jax.ShapeDtypeStruct((total, HIDDEN), jnp.float32),
        scratch_types=[
            pltpu.VMEM((2 * CHUNK,), jnp.float32),
            pltpu.VMEM((2, CHUNK, HIDDEN), jnp.float32),
            pltpu.VMEM((N_BINS * HIDDEN,), jnp.float32),
            pltpu.VMEM((HP_LEN,), jnp.float32),
            pltpu.SemaphoreType.DMA,
            pltpu.SemaphoreType.DMA,
        ],
    )
    def sc_kernel(x_hbm, hp_hbm, table_hbm, out_hbm, x_v, rows_v, tab_v, hp_v,
                  xsem, ssem):
        wid = lax.axis_index("s") * _NC + lax.axis_index("c")
        wbase = wid * per_w

        pltpu.sync_copy(hp_hbm, hp_v)
        pltpu.sync_copy(table_hbm, tab_v)
        pltpu.async_copy(
            x_hbm.at[pl.ds(pl.multiple_of(wbase, CHUNK), CHUNK)],
            x_v.at[pl.ds(0, CHUNK)],
            xsem,
        )

        lane = lax.iota(jnp.int32, 16)

        def ebase(t):
            return pl.multiple_of(wbase + t * CHUNK, CHUNK)

        def wait_store(b):
            pltpu.make_async_copy(
                rows_v.at[b], out_hbm.at[pl.ds(0, CHUNK)], ssem
            ).wait()

        def group_body(g, carry):
            for b in range(2):
                t = g * 2 + b

                pl.when(g >= 1)(functools.partial(wait_store, b))

                pltpu.make_async_copy(
                    x_hbm.at[pl.ds(0, CHUNK)], x_v.at[pl.ds(b * CHUNK, CHUNK)], xsem
                ).wait()

                def prefetch():
                    pltpu.async_copy(
                        x_hbm.at[pl.ds(ebase(t + 1), CHUNK)],
                        x_v.at[pl.ds((1 - b) * CHUNK, CHUNK)],
                        xsem,
                    )
                if b == 0:
                    prefetch()
                else:
                    pl.when(g < n_groups - 1)(prefetch)

                def extract(blk):
                    xv = x_v[pl.ds(b * CHUNK + blk * 16, 16)]
                    t0 = (xv - MIN_VAL) * SCALE
                    gi = jnp.clip(t0.astype(jnp.int32), 0, N_BINS - 1)
                    hi = plsc.load_gather(hp_v, [gi + 1])
                    lo = plsc.load_gather(hp_v, [gi])
                    gi = gi + jnp.where(xv > hi, 1, 0) - jnp.where(xv <= lo, 1, 0)
                    gofs = gi * HIDDEN
                    return [gofs[e] for e in range(16)]

                def copy_rows(blk, bases):
                    for e in range(16):
                        row = blk * 16 + e
                        for k in range(0, HIDDEN, 16):
                            rows_v[b, row, pl.ds(k, 16)] = (
                                tab_v[pl.ds(bases[e] + k, 16)]
                            )

                bases = extract(0)
                for blk in range(1, CHUNK // 16):
                    nxt = extract(blk)
                    copy_rows(blk - 1, bases)
                    bases = nxt
                copy_rows(CHUNK // 16 - 1, bases)

                pltpu.async_copy(
                    rows_v.at[b], out_hbm.at[pl.ds(ebase(t), CHUNK)], ssem
                )
            return carry

        lax.fori_loop(0, n_groups, group_body, 0)

        wait_store(0)
        wait_store(1)

    return sc_kernel


def kernel(x, boundaries, table):
    n_rows, row_len = x.shape
    total = n_rows * row_len
    xf = x.reshape(total)
    hp = jnp.concatenate(
        [
            jnp.full((1,), -jnp.inf, jnp.float32),
            boundaries.astype(jnp.float32),
            jnp.full((HP_LEN - 1 - boundaries.shape[0],), jnp.inf, jnp.float32),
        ]
    )
    tab_flat = table.reshape(N_BINS * HIDDEN)
    out = _make_sc_call(total)(xf, hp, tab_flat)
    return out.reshape(n_rows, row_len, HIDDEN)
